# baseline (speedup 1.0000x reference)
import functools

import jax
import jax.numpy as jnp
from jax import lax
from jax.experimental import pallas as pl
from jax.experimental.pallas import tpu as pltpu
from jax.experimental.pallas import tpu_sc as plsc

NU = 5000
NI = 5000
NN = NU + NI
NE = 160000
FD = 128
EPS = 0.1

NS = 16
EPT = NE // NS
CH = 125
NCHUNK = EPT // CH

ZCH = 80
NZFULL = NU // ZCH
ZREM = NU - NZFULL * ZCH
ZREM_TILE = NZFULL % NS

_F32 = jnp.float32


@functools.partial(
    pl.kernel,
    out_type=jax.ShapeDtypeStruct((NN,), _F32),
    mesh=plsc.VectorSubcoreMesh(core_axis_name="c", subcore_axis_name="s"),
    scratch_types=[
        pltpu.VMEM((NCHUNK, CH), jnp.int32),
        pltpu.VMEM((128,), _F32),
        pltpu.VMEM((320,), _F32),
        pltpu.VMEM_SHARED((NU,), _F32),
    ],
)
def _deg_kernel(eu3_hbm, ei3_hbm, deg_hbm, idx_v, ones_v, zrow_v, acc):
    c = lax.axis_index("c")
    s = lax.axis_index("s")

    @pl.when(c == 0)
    def _():
        pltpu.sync_copy(eu3_hbm.at[s], idx_v)

    @pl.when(c == 1)
    def _():
        pltpu.sync_copy(ei3_hbm.at[s], idx_v)

    @pl.loop(0, 128 // 16)
    def _(i):
        ones_v[pl.ds(i * 16, 16)] = jnp.ones((16,), _F32)

    @pl.loop(0, 320 // 16)
    def _(i):
        zrow_v[pl.ds(i * 16, 16)] = jnp.zeros((16,), _F32)

    @pl.when(s < 15)
    def _():
        pltpu.sync_copy(zrow_v, acc.at[pl.ds(s * 320, 320)])

    @pl.when(s == 15)
    def _():
        pltpu.sync_copy(zrow_v.at[pl.ds(0, 200)], acc.at[pl.ds(4800, 200)])

    plsc.subcore_barrier()

    @pl.loop(0, NCHUNK)
    def _(j):
        pltpu.sync_copy(ones_v.at[pl.ds(0, CH)], acc.at[idx_v.at[j]],
                        add=True)

    plsc.subcore_barrier()

    @pl.when(s < 15)
    def _():
        pltpu.sync_copy(acc.at[pl.ds(s * 320, 320)], zrow_v)
        pltpu.sync_copy(zrow_v, deg_hbm.at[pl.ds(c * NU + s * 320, 320)])

    @pl.when(s == 15)
    def _():
        pltpu.sync_copy(acc.at[pl.ds(4800, 200)], zrow_v.at[pl.ds(0, 200)])
        pltpu.sync_copy(zrow_v.at[pl.ds(0, 200)],
                        deg_hbm.at[pl.ds(c * NU + 4800, 200)])


_OUT6 = [jax.ShapeDtypeStruct((NU, FD), _F32) for _ in range(6)]


@functools.partial(
    pl.kernel,
    out_type=_OUT6,
    mesh=plsc.VectorSubcoreMesh(core_axis_name="c", subcore_axis_name="s"),
    scratch_types=[
        pltpu.VMEM((NCHUNK, CH), jnp.int32),
        pltpu.VMEM((NCHUNK, CH), jnp.int32),
        pltpu.VMEM((CH, FD), _F32),
        pltpu.VMEM((CH, FD), _F32),
        pltpu.VMEM((CH, FD), _F32),
        pltpu.VMEM((CH, FD), _F32),
        pltpu.VMEM((ZCH // 4, FD), _F32),
        pltpu.VMEM_SHARED((NU, FD), _F32),
        pltpu.SemaphoreType.DMA,
        pltpu.SemaphoreType.DMA,
        pltpu.SemaphoreType.DMA,
        pltpu.SemaphoreType.DMA,
    ],
)
def _spmm_kernel(fu0_hbm, fu1_hbm, fu2_hbm, fi0_hbm, fi1_hbm, fi2_hbm,
                 eu3_hbm, ei3_hbm,
                 o0u_hbm, o1u_hbm, o2u_hbm, o0i_hbm, o1i_hbm, o2i_hbm,
                 sidx_v, didx_v, buf0_v, buf1_v, buf2_v, buf3_v, zbuf_v, acc,
                 sem0, sem1, sem2, sem3):
    c = lax.axis_index("c")
    s = lax.axis_index("s")

    @pl.loop(0, ZCH // 4)
    def _(i):
        @pl.loop(0, FD // 16)
        def _(j):
            zbuf_v[i, pl.ds(j * 16, 16)] = jnp.zeros((16,), _F32)

    def zero_rows(base, n_quarters):
        for q in range(n_quarters):
            pltpu.sync_copy(zbuf_v,
                            acc.at[pl.ds(base + q * (ZCH // 4), ZCH // 4)])

    def zero_chunk(base):
        zero_rows(base, 4)

    def run_side(sidx_hbm, didx_hbm, feats, outs):
        pltpu.sync_copy(sidx_hbm.at[s], sidx_v)
        pltpu.sync_copy(didx_hbm.at[s], didx_v)

        @pl.loop(0, NZFULL)
        def _(k):
            @pl.when(lax.rem(k, NS) == s)
            def _():
                zero_chunk(k * ZCH)

        @pl.when(s == ZREM_TILE)
        def _():
            zero_rows(NZFULL * ZCH, 2)

        plsc.subcore_barrier()

        for pass_no, (feat_hbm, out_hbm) in enumerate(zip(feats, outs)):

            def gsrc(j):
                return feat_hbm.at[sidx_v.at[j]]

            bufs4 = (buf0_v, buf1_v, buf2_v, buf3_v)
            gsems = (sem0, sem1, sem2, sem3)
            pltpu.async_copy(gsrc(0), buf0_v, sem0)
            pltpu.async_copy(gsrc(1), buf1_v, sem1)
            pltpu.async_copy(gsrc(2), buf2_v, sem2)

            @pl.loop(0, NCHUNK // 4 - 1)
            def _(k):
                for d in range(4):
                    j = 4 * k + d
                    buf, sem = bufs4[d], gsems[d]
                    nbuf, nsem = bufs4[(d + 3) % 4], gsems[(d + 3) % 4]
                    pltpu.make_async_copy(gsrc(j), buf, sem).wait()
                    pltpu.async_copy(gsrc(j + 3), nbuf, nsem)
                    pltpu.sync_copy(buf, acc.at[didx_v.at[j]], add=True)

            pltpu.make_async_copy(gsrc(NCHUNK - 4), buf0_v, sem0).wait()
            pltpu.async_copy(gsrc(NCHUNK - 1), buf3_v, sem3)
            pltpu.sync_copy(buf0_v, acc.at[didx_v.at[NCHUNK - 4]], add=True)
            pltpu.make_async_copy(gsrc(NCHUNK - 3), buf1_v, sem1).wait()
            pltpu.sync_copy(buf1_v, acc.at[didx_v.at[NCHUNK - 3]], add=True)
            pltpu.make_async_copy(gsrc(NCHUNK - 2), buf2_v, sem2).wait()
            pltpu.sync_copy(buf2_v, acc.at[didx_v.at[NCHUNK - 2]], add=True)
            pltpu.make_async_copy(gsrc(NCHUNK - 1), buf3_v, sem3).wait()
            pltpu.sync_copy(buf3_v, acc.at[didx_v.at[NCHUNK - 1]], add=True)

            plsc.subcore_barrier()

            last_pass = pass_no == len(feats) - 1
            bufs = (buf0_v, buf1_v, buf0_v, buf1_v)
            sems = (sem0, sem1, sem0, sem1)
            for t in range(4):
                k = s + NS * t
                buf, sem = bufs[t], sems[t]

                @pl.when(k < NZFULL)
                def _():
                    if t >= 2:
                        pltpu.make_async_copy(
                            buf.at[pl.ds(0, ZCH)],
                            out_hbm.at[pl.ds((k - 2 * NS) * ZCH, ZCH)],
                            sem).wait()
                    pltpu.sync_copy(acc.at[pl.ds(k * ZCH, ZCH)],
                                    buf.at[pl.ds(0, ZCH)])
                    pltpu.async_copy(buf.at[pl.ds(0, ZCH)],
                                     out_hbm.at[pl.ds(k * ZCH, ZCH)], sem)
                    if not last_pass:
                        zero_chunk(k * ZCH)

            @pl.when(s == ZREM_TILE)
            def _():
                pltpu.make_async_copy(
                    buf0_v.at[pl.ds(0, ZCH)],
                    out_hbm.at[pl.ds((ZREM_TILE + 2 * NS) * ZCH, ZCH)],
                    sem0).wait()
                pltpu.sync_copy(acc.at[pl.ds(NZFULL * ZCH, ZREM)],
                                buf0_v.at[pl.ds(0, ZREM)])
                pltpu.async_copy(buf0_v.at[pl.ds(0, ZREM)],
                                 out_hbm.at[pl.ds(NZFULL * ZCH, ZREM)], sem0)
                if not last_pass:
                    zero_rows(NZFULL * ZCH, 2)

            @pl.when((s + NS * 2 < NZFULL) & (s != ZREM_TILE))
            def _():
                pltpu.make_async_copy(
                    buf0_v.at[pl.ds(0, ZCH)],
                    out_hbm.at[pl.ds((s + 2 * NS) * ZCH, ZCH)], sem0).wait()

            @pl.when(s == ZREM_TILE)
            def _():
                pltpu.make_async_copy(
                    buf0_v.at[pl.ds(0, ZREM)],
                    out_hbm.at[pl.ds(NZFULL * ZCH, ZREM)], sem0).wait()

            @pl.when(s + NS * 3 < NZFULL)
            def _():
                pltpu.make_async_copy(
                    buf1_v.at[pl.ds(0, ZCH)],
                    out_hbm.at[pl.ds((s + 3 * NS) * ZCH, ZCH)], sem1).wait()

            @pl.when((s + NS * 3 >= NZFULL) & (s + NS < NZFULL))
            def _():
                pltpu.make_async_copy(
                    buf1_v.at[pl.ds(0, ZCH)],
                    out_hbm.at[pl.ds((s + NS) * ZCH, ZCH)], sem1).wait()

            plsc.subcore_barrier()

    @pl.when(c == 0)
    def _():
        run_side(ei3_hbm, eu3_hbm, (fi0_hbm, fi1_hbm, fi2_hbm),
                 (o0u_hbm, o1u_hbm, o2u_hbm))

    @pl.when(c == 1)
    def _():
        run_side(eu3_hbm, ei3_hbm, (fu0_hbm, fu1_hbm, fu2_hbm),
                 (o0i_hbm, o1i_hbm, o2i_hbm))


_RB = 2000


_RBH = 1000


def _uspec(block):
    return pl.BlockSpec(block, lambda i: (i, 0))


def _ispec(block):
    return pl.BlockSpec(block, lambda i: (i + NU // _RBH, 0))


def _prep_body(degu_ref, degi_ref, xu_ref, xi_ref, r1u_ref, r1i_ref,
               r2u_ref, r2i_ref,
               fu0_ref, fu1_ref, fu2_ref, fi0_ref, fi1_ref, fi2_ref):
    for (deg_ref, x_ref, r1_ref, r2_ref, f0_ref, f1_ref, f2_ref) in (
            (degu_ref, xu_ref, r1u_ref, r2u_ref, fu0_ref, fu1_ref, fu2_ref),
            (degi_ref, xi_ref, r1i_ref, r2i_ref, fi0_ref, fi1_ref, fi2_ref)):
        deg = deg_ref[...]
        d_inv = jnp.where(deg > 0.0, lax.rsqrt(deg), 0.0)
        xb = x_ref[...]
        sx = jnp.sign(xb)
        p1 = xb + r1_ref[...] * sx * EPS
        p2 = xb + r2_ref[...] * sx * EPS
        f0_ref[...] = xb * d_inv
        f1_ref[...] = p1 * d_inv
        f2_ref[...] = p2 * d_inv


_prep = pl.pallas_call(
    _prep_body,
    grid=(NU // _RBH,),
    in_specs=[
        _uspec((_RBH, 1)), _ispec((_RBH, 1)),
        _uspec((_RBH, FD)), _ispec((_RBH, FD)),
        _uspec((_RBH, FD)), _ispec((_RBH, FD)),
        _uspec((_RBH, FD)), _ispec((_RBH, FD)),
    ],
    out_specs=[_uspec((_RBH, FD)) for _ in range(6)],
    out_shape=[jax.ShapeDtypeStruct((NU, FD), _F32) for _ in range(6)],
)


def _post_body(degu_ref, degi_ref, xu_ref, xi_ref, r1u_ref, r1i_ref,
               r2u_ref, r2i_ref, a0u_ref, a0i_ref, a1u_ref, a1i_ref,
               a2u_ref, a2i_ref,
               ua_ref, ia_ref, uap1_ref, iap1_ref, uap2_ref, iap2_ref,
               g0u_ref, g0i_ref):
    for (deg_ref, x_ref, r1_ref, r2_ref, a0_ref, a1_ref, a2_ref,
         ae_ref, ap1_ref, ap2_ref, g_ref) in (
            (degu_ref, xu_ref, r1u_ref, r2u_ref, a0u_ref, a1u_ref, a2u_ref,
             ua_ref, uap1_ref, uap2_ref, g0u_ref),
            (degi_ref, xi_ref, r1i_ref, r2i_ref, a0i_ref, a1i_ref, a2i_ref,
             ia_ref, iap1_ref, iap2_ref, g0i_ref)):
        deg = deg_ref[...]
        d_inv = jnp.where(deg > 0.0, lax.rsqrt(deg), 0.0)
        xb = x_ref[...]
        sx = jnp.sign(xb)
        p1 = xb + r1_ref[...] * sx * EPS
        p2 = xb + r2_ref[...] * sx * EPS
        g0 = a0_ref[...] * d_inv
        g1 = a1_ref[...] * d_inv
        g2 = a2_ref[...] * d_inv
        ae_ref[...] = 2.0 * xb + g0
        ap1_ref[...] = 2.0 * p1 + g1
        ap2_ref[...] = 2.0 * p2 + g2
        g_ref[...] = g0


_post = pl.pallas_call(
    _post_body,
    grid=(NU // _RBH,),
    in_specs=[
        _uspec((_RBH, 1)), _ispec((_RBH, 1)),
        _uspec((_RBH, FD)), _ispec((_RBH, FD)),
        _uspec((_RBH, FD)), _ispec((_RBH, FD)),
        _uspec((_RBH, FD)), _ispec((_RBH, FD)),
        _uspec((_RBH, FD)), _uspec((_RBH, FD)),
        _uspec((_RBH, FD)), _uspec((_RBH, FD)),
        _uspec((_RBH, FD)), _uspec((_RBH, FD)),
    ],
    out_specs=[_uspec((_RBH, FD)) for _ in range(8)],
    out_shape=[jax.ShapeDtypeStruct((NU, FD), _F32) for _ in range(8)],
)


def kernel(x, rand1, rand2, edge_u, edge_i):
    eu = edge_u.astype(jnp.int32)
    ei = edge_i.astype(jnp.int32)
    eu3 = eu.reshape(NS, NCHUNK, CH)
    ei3 = ei.reshape(NS, NCHUNK, CH)

    deg = _deg_kernel(eu3, ei3)
    deg2 = deg.reshape(NN, 1)
    fu0, fu1, fu2, fi0, fi1, fi2 = _prep(deg2, deg2, x, x, rand1, rand1,
                                         rand2, rand2)
    (a0u, a1u, a2u, a0i, a1i, a2i) = _spmm_kernel(
        fu0, fu1, fu2, fi0, fi1, fi2, eu3, ei3)
    (ua, ia, uap1, iap1, uap2, iap2, g0u, g0i) = _post(
        deg2, deg2, x, x, rand1, rand1, rand2, rand2,
        a0u, a0i, a1u, a1i, a2u, a2i)
    return (ua, ia, uap1, iap1, uap2, iap2,
            jnp.concatenate([g0u, g0i], axis=0))


# speedup vs baseline: 1.0116x; 1.0116x over previous
import functools

import jax
import jax.numpy as jnp
from jax import lax
from jax.experimental import pallas as pl
from jax.experimental.pallas import tpu as pltpu
from jax.experimental.pallas import tpu_sc as plsc

NU = 5000
NI = 5000
NN = NU + NI
NE = 160000
FD = 128
EPS = 0.1

NS = 16
EPT = NE // NS
CH = 125
NCHUNK = EPT // CH

ZCH = 80
NZFULL = NU // ZCH
ZREM = NU - NZFULL * ZCH
ZREM_TILE = NZFULL % NS

_F32 = jnp.float32


@functools.partial(
    pl.kernel,
    out_type=jax.ShapeDtypeStruct((NN,), _F32),
    mesh=plsc.VectorSubcoreMesh(core_axis_name="c", subcore_axis_name="s"),
    scratch_types=[
        pltpu.VMEM((NCHUNK, CH), jnp.int32),
        pltpu.VMEM((128,), _F32),
        pltpu.VMEM((320,), _F32),
        pltpu.VMEM_SHARED((NU,), _F32),
    ],
)
def _deg_kernel(eu3_hbm, ei3_hbm, deg_hbm, idx_v, ones_v, zrow_v, acc):
    c = lax.axis_index("c")
    s = lax.axis_index("s")

    @pl.when(c == 0)
    def _():
        pltpu.sync_copy(eu3_hbm.at[s], idx_v)

    @pl.when(c == 1)
    def _():
        pltpu.sync_copy(ei3_hbm.at[s], idx_v)

    @pl.loop(0, 128 // 16)
    def _(i):
        ones_v[pl.ds(i * 16, 16)] = jnp.ones((16,), _F32)

    @pl.loop(0, 320 // 16)
    def _(i):
        zrow_v[pl.ds(i * 16, 16)] = jnp.zeros((16,), _F32)

    @pl.when(s < 15)
    def _():
        pltpu.sync_copy(zrow_v, acc.at[pl.ds(s * 320, 320)])

    @pl.when(s == 15)
    def _():
        pltpu.sync_copy(zrow_v.at[pl.ds(0, 200)], acc.at[pl.ds(4800, 200)])

    plsc.subcore_barrier()

    @pl.loop(0, NCHUNK)
    def _(j):
        pltpu.sync_copy(ones_v.at[pl.ds(0, CH)], acc.at[idx_v.at[j]],
                        add=True)

    plsc.subcore_barrier()

    @pl.when(s < 15)
    def _():
        pltpu.sync_copy(acc.at[pl.ds(s * 320, 320)], zrow_v)
        pltpu.sync_copy(zrow_v, deg_hbm.at[pl.ds(c * NU + s * 320, 320)])

    @pl.when(s == 15)
    def _():
        pltpu.sync_copy(acc.at[pl.ds(4800, 200)], zrow_v.at[pl.ds(0, 200)])
        pltpu.sync_copy(zrow_v.at[pl.ds(0, 200)],
                        deg_hbm.at[pl.ds(c * NU + 4800, 200)])


_OUT6 = [jax.ShapeDtypeStruct((NU, FD), _F32) for _ in range(6)]


@functools.partial(
    pl.kernel,
    out_type=_OUT6,
    mesh=plsc.VectorSubcoreMesh(core_axis_name="c", subcore_axis_name="s"),
    scratch_types=[
        pltpu.VMEM((NCHUNK, CH), jnp.int32),
        pltpu.VMEM((NCHUNK, CH), jnp.int32),
        pltpu.VMEM((CH, FD), _F32),
        pltpu.VMEM((CH, FD), _F32),
        pltpu.VMEM((CH, FD), _F32),
        pltpu.VMEM((ZCH, FD), _F32),
        pltpu.VMEM_SHARED((NU, FD), _F32),
        pltpu.SemaphoreType.DMA,
        pltpu.SemaphoreType.DMA,
        pltpu.SemaphoreType.DMA,
    ],
)
def _spmm_kernel(fu0_hbm, fu1_hbm, fu2_hbm, fi0_hbm, fi1_hbm, fi2_hbm,
                 eu3_hbm, ei3_hbm,
                 o0u_hbm, o1u_hbm, o2u_hbm, o0i_hbm, o1i_hbm, o2i_hbm,
                 sidx_v, didx_v, buf0_v, buf1_v, buf2_v, zbuf_v, acc,
                 sem0, sem1, sem2):
    c = lax.axis_index("c")
    s = lax.axis_index("s")

    @pl.loop(0, ZCH)
    def _(i):
        @pl.loop(0, FD // 16)
        def _(j):
            zbuf_v[i, pl.ds(j * 16, 16)] = jnp.zeros((16,), _F32)

    def zero_chunk(base):
        pltpu.sync_copy(zbuf_v, acc.at[pl.ds(base, ZCH)])

    def zero_rem(base):
        pltpu.sync_copy(zbuf_v.at[pl.ds(0, ZREM)], acc.at[pl.ds(base, ZREM)])

    def run_side(sidx_hbm, didx_hbm, feats, outs):
        pltpu.sync_copy(sidx_hbm.at[s], sidx_v)
        pltpu.sync_copy(didx_hbm.at[s], didx_v)

        @pl.loop(0, NZFULL)
        def _(k):
            @pl.when(lax.rem(k, NS) == s)
            def _():
                zero_chunk(k * ZCH)

        @pl.when(s == ZREM_TILE)
        def _():
            zero_rem(NZFULL * ZCH)

        plsc.subcore_barrier()

        for pass_no, (feat_hbm, out_hbm) in enumerate(zip(feats, outs)):

            def gsrc(j):
                return feat_hbm.at[sidx_v.at[j]]

            bufs3 = (buf0_v, buf1_v, buf2_v)
            gsems = (sem0, sem1, sem2)
            pltpu.async_copy(gsrc(0), buf0_v, sem0)
            pltpu.async_copy(gsrc(1), buf1_v, sem1)

            @pl.loop(0, NCHUNK // 3)
            def _(k):
                for d in range(3):
                    j = 3 * k + d
                    buf, sem = bufs3[d], gsems[d]
                    nbuf, nsem = bufs3[(d + 2) % 3], gsems[(d + 2) % 3]
                    pltpu.make_async_copy(gsrc(j), buf, sem).wait()
                    pltpu.async_copy(gsrc(j + 2), nbuf, nsem)
                    pltpu.sync_copy(buf, acc.at[didx_v.at[j]], add=True)

            pltpu.make_async_copy(gsrc(NCHUNK - 2), buf0_v, sem0).wait()
            pltpu.sync_copy(buf0_v, acc.at[didx_v.at[NCHUNK - 2]], add=True)
            pltpu.make_async_copy(gsrc(NCHUNK - 1), buf1_v, sem1).wait()
            pltpu.sync_copy(buf1_v, acc.at[didx_v.at[NCHUNK - 1]], add=True)

            plsc.subcore_barrier()

            last_pass = pass_no == len(feats) - 1
            bufs = (buf0_v, buf1_v, buf0_v, buf1_v)
            sems = (sem0, sem1, sem0, sem1)
            for t in range(4):
                k = s + NS * t
                buf, sem = bufs[t], sems[t]

                @pl.when(k < NZFULL)
                def _():
                    if t >= 2:
                        pltpu.make_async_copy(
                            buf.at[pl.ds(0, ZCH)],
                            out_hbm.at[pl.ds((k - 2 * NS) * ZCH, ZCH)],
                            sem).wait()
                    pltpu.sync_copy(acc.at[pl.ds(k * ZCH, ZCH)],
                                    buf.at[pl.ds(0, ZCH)])
                    pltpu.async_copy(buf.at[pl.ds(0, ZCH)],
                                     out_hbm.at[pl.ds(k * ZCH, ZCH)], sem)
                    if not last_pass:
                        zero_chunk(k * ZCH)

            @pl.when(s == ZREM_TILE)
            def _():
                pltpu.make_async_copy(
                    buf0_v.at[pl.ds(0, ZCH)],
                    out_hbm.at[pl.ds((ZREM_TILE + 2 * NS) * ZCH, ZCH)],
                    sem0).wait()
                pltpu.sync_copy(acc.at[pl.ds(NZFULL * ZCH, ZREM)],
                                buf0_v.at[pl.ds(0, ZREM)])
                pltpu.async_copy(buf0_v.at[pl.ds(0, ZREM)],
                                 out_hbm.at[pl.ds(NZFULL * ZCH, ZREM)], sem0)
                if not last_pass:
                    zero_rem(NZFULL * ZCH)

            @pl.when((s + NS * 2 < NZFULL) & (s != ZREM_TILE))
            def _():
                pltpu.make_async_copy(
                    buf0_v.at[pl.ds(0, ZCH)],
                    out_hbm.at[pl.ds((s + 2 * NS) * ZCH, ZCH)], sem0).wait()

            @pl.when(s == ZREM_TILE)
            def _():
                pltpu.make_async_copy(
                    buf0_v.at[pl.ds(0, ZREM)],
                    out_hbm.at[pl.ds(NZFULL * ZCH, ZREM)], sem0).wait()

            @pl.when(s + NS * 3 < NZFULL)
            def _():
                pltpu.make_async_copy(
                    buf1_v.at[pl.ds(0, ZCH)],
                    out_hbm.at[pl.ds((s + 3 * NS) * ZCH, ZCH)], sem1).wait()

            @pl.when((s + NS * 3 >= NZFULL) & (s + NS < NZFULL))
            def _():
                pltpu.make_async_copy(
                    buf1_v.at[pl.ds(0, ZCH)],
                    out_hbm.at[pl.ds((s + NS) * ZCH, ZCH)], sem1).wait()

            plsc.subcore_barrier()

    @pl.when(c == 0)
    def _():
        run_side(ei3_hbm, eu3_hbm, (fi0_hbm, fi1_hbm, fi2_hbm),
                 (o0u_hbm, o1u_hbm, o2u_hbm))

    @pl.when(c == 1)
    def _():
        run_side(eu3_hbm, ei3_hbm, (fu0_hbm, fu1_hbm, fu2_hbm),
                 (o0i_hbm, o1i_hbm, o2i_hbm))


_RB = 2000


_RBH = 1000


def _uspec(block):
    return pl.BlockSpec(block, lambda i: (i, 0))


def _ispec(block):
    return pl.BlockSpec(block, lambda i: (i + NU // _RBH, 0))


def _prep_body(degu_ref, degi_ref, xu_ref, xi_ref, r1u_ref, r1i_ref,
               r2u_ref, r2i_ref,
               fu0_ref, fu1_ref, fu2_ref, fi0_ref, fi1_ref, fi2_ref):
    for (deg_ref, x_ref, r1_ref, r2_ref, f0_ref, f1_ref, f2_ref) in (
            (degu_ref, xu_ref, r1u_ref, r2u_ref, fu0_ref, fu1_ref, fu2_ref),
            (degi_ref, xi_ref, r1i_ref, r2i_ref, fi0_ref, fi1_ref, fi2_ref)):
        deg = deg_ref[...]
        d_inv = jnp.where(deg > 0.0, lax.rsqrt(deg), 0.0)
        xb = x_ref[...]
        sx = jnp.sign(xb)
        p1 = xb + r1_ref[...] * sx * EPS
        p2 = xb + r2_ref[...] * sx * EPS
        f0_ref[...] = xb * d_inv
        f1_ref[...] = p1 * d_inv
        f2_ref[...] = p2 * d_inv


_prep = pl.pallas_call(
    _prep_body,
    grid=(NU // _RBH,),
    in_specs=[
        _uspec((_RBH, 1)), _ispec((_RBH, 1)),
        _uspec((_RBH, FD)), _ispec((_RBH, FD)),
        _uspec((_RBH, FD)), _ispec((_RBH, FD)),
        _uspec((_RBH, FD)), _ispec((_RBH, FD)),
    ],
    out_specs=[_uspec((_RBH, FD)) for _ in range(6)],
    out_shape=[jax.ShapeDtypeStruct((NU, FD), _F32) for _ in range(6)],
)


def _post_body(degu_ref, degi_ref, xu_ref, xi_ref, r1u_ref, r1i_ref,
               r2u_ref, r2i_ref, a0u_ref, a0i_ref, a1u_ref, a1i_ref,
               a2u_ref, a2i_ref,
               ua_ref, ia_ref, uap1_ref, iap1_ref, uap2_ref, iap2_ref,
               g0u_ref, g0i_ref):
    for (deg_ref, x_ref, r1_ref, r2_ref, a0_ref, a1_ref, a2_ref,
         ae_ref, ap1_ref, ap2_ref, g_ref) in (
            (degu_ref, xu_ref, r1u_ref, r2u_ref, a0u_ref, a1u_ref, a2u_ref,
             ua_ref, uap1_ref, uap2_ref, g0u_ref),
            (degi_ref, xi_ref, r1i_ref, r2i_ref, a0i_ref, a1i_ref, a2i_ref,
             ia_ref, iap1_ref, iap2_ref, g0i_ref)):
        deg = deg_ref[...]
        d_inv = jnp.where(deg > 0.0, lax.rsqrt(deg), 0.0)
        xb = x_ref[...]
        sx = jnp.sign(xb)
        p1 = xb + r1_ref[...] * sx * EPS
        p2 = xb + r2_ref[...] * sx * EPS
        g0 = a0_ref[...] * d_inv
        g1 = a1_ref[...] * d_inv
        g2 = a2_ref[...] * d_inv
        ae_ref[...] = 2.0 * xb + g0
        ap1_ref[...] = 2.0 * p1 + g1
        ap2_ref[...] = 2.0 * p2 + g2
        g_ref[...] = g0


_post = pl.pallas_call(
    _post_body,
    grid=(NU // _RBH,),
    in_specs=[
        _uspec((_RBH, 1)), _ispec((_RBH, 1)),
        _uspec((_RBH, FD)), _ispec((_RBH, FD)),
        _uspec((_RBH, FD)), _ispec((_RBH, FD)),
        _uspec((_RBH, FD)), _ispec((_RBH, FD)),
        _uspec((_RBH, FD)), _uspec((_RBH, FD)),
        _uspec((_RBH, FD)), _uspec((_RBH, FD)),
        _uspec((_RBH, FD)), _uspec((_RBH, FD)),
    ],
    out_specs=[_uspec((_RBH, FD)) for _ in range(8)],
    out_shape=[jax.ShapeDtypeStruct((NU, FD), _F32) for _ in range(8)],
)


def kernel(x, rand1, rand2, edge_u, edge_i):
    eu = edge_u.astype(jnp.int32)
    ei = edge_i.astype(jnp.int32)
    eu3 = eu.reshape(NS, NCHUNK, CH)
    ei3 = ei.reshape(NS, NCHUNK, CH)

    deg = _deg_kernel(eu3, ei3)
    deg2 = deg.reshape(NN, 1)
    fu0, fu1, fu2, fi0, fi1, fi2 = _prep(deg2, deg2, x, x, rand1, rand1,
                                         rand2, rand2)
    (a0u, a1u, a2u, a0i, a1i, a2i) = _spmm_kernel(
        fu0, fu1, fu2, fi0, fi1, fi2, eu3, ei3)
    (ua, ia, uap1, iap1, uap2, iap2, g0u, g0i) = _post(
        deg2, deg2, x, x, rand1, rand1, rand2, rand2,
        a0u, a0i, a1u, a1i, a2u, a2i)
    return (ua, ia, uap1, iap1, uap2, iap2,
            jnp.concatenate([g0u, g0i], axis=0))
